# baseline (speedup 1.0000x reference)
import functools

import jax
import jax.numpy as jnp
from jax import lax
from jax.experimental import pallas as pl
from jax.experimental.pallas import tpu as pltpu
from jax.experimental.pallas import tpu_sc as plsc

H, W = 1080, 1920
C = 3
L = 16
RING = 32
SW = 640
CW = 896
JG = SW // L
NC, NS = 2, 16
NW = NC * NS
NGRP = 6
TOL = 0.05

LOG2E = 1.4426950408889634
CE = (0.9999961199303905, 0.6930292690219008, 0.23938500062629817,
      0.05318647048254017, 0.006838262187515184)
CL = (2.2132784000594707e-05, 0.9990102089269637, -0.4891557820114497,
      0.28330238362040977, -0.1301179302884552, 0.030102247599666062)


def _softplus_terms(tg, tp, sg, sgx, sp):
    tgp = tg + 1e-8
    pos = sg >= (1.0 + TOL) * tgp
    neg = sgx <= tgp
    nz = pos | neg
    diff = sp - tp
    ad = jnp.abs(diff)
    x = jnp.maximum(ad * (-LOG2E), -126.0)
    ki = x.astype(jnp.int32)
    fr = x - ki.astype(jnp.float32)
    p2 = jnp.float32(CE[4])
    for c in (CE[3], CE[2], CE[1], CE[0]):
        p2 = p2 * fr + c
    scale = plsc.bitcast((ki + 127) << 23, jnp.float32)
    e = p2 * scale
    l = jnp.float32(CL[5])
    for c in (CL[4], CL[3], CL[2], CL[1], CL[0]):
        l = l * e + c
    t = jnp.where(pos, -diff, diff)
    soft = jnp.maximum(t, 0.0) + l
    m = jnp.where(nz, 1.0, 0.0)
    s = jnp.where(nz, soft, 0.0)
    q = jnp.where(nz, 0.0, diff * diff)
    return m, s, q


def _sc_body(gs_hbm, gt_hbm, pr_hbm, out_hbm, gtb, prb, gxa, gya,
             gxb, gyb, accb, sema, semb, semg):
    cid = lax.axis_index("c")
    sid = lax.axis_index("s")
    wid = sid * NC + cid

    lo = (wid * H) // NW
    hi = ((wid + 1) * H) // NW
    glo = lo // 8

    zero = jnp.zeros((L,), jnp.float32)

    def strip_pass(s, acc):
        cs = pl.multiple_of(s * 512, 128)
        sloc = s * 128
        scol = pl.multiple_of(s * SW, 128)

        def ring_dma(g):
            g = pl.multiple_of(g, 8)
            slot = pl.multiple_of(g & (RING - 1), 8)
            return [
                pltpu.make_async_copy(gt_hbm.at[pl.ds(g, 8), pl.ds(cs, CW)],
                                      gtb.at[pl.ds(slot, 8)], semg),
                pltpu.make_async_copy(pr_hbm.at[pl.ds(g, 8), pl.ds(cs, CW)],
                                      prb.at[pl.ds(slot, 8)], semg),
            ]

        def load_ring_group(g):
            for d in ring_dma(g):
                d.start()
            for d in ring_dma(g):
                d.wait()

        def idx_dma(gi, bufx, bufy, sem):
            g8 = pl.multiple_of(jnp.minimum(gi * 8, H - 8), 8)
            ds = []
            for c in range(C):
                ds.append(pltpu.make_async_copy(
                    gs_hbm.at[0, c, pl.ds(g8, 8), pl.ds(scol, SW)],
                    bufx.at[c], sem))
                ds.append(pltpu.make_async_copy(
                    gs_hbm.at[1, c, pl.ds(g8, 8), pl.ds(scol, SW)],
                    bufy.at[c], sem))
            return ds

        def idx_start(gi, bufx, bufy, sem):
            for d in idx_dma(gi, bufx, bufy, sem):
                d.start()

        def idx_wait(gi, bufx, bufy, sem):
            for d in idx_dma(gi, bufx, bufy, sem):
                d.wait()

        pg0 = jnp.maximum(lo - 10, 0) // 8
        pmax = (lo + 13) // 8

        def pre_body(gi, _):
            load_ring_group(gi * 8)
            return 0

        lax.fori_loop(pg0, pmax + 1, pre_body, 0)

        def process_group(gi, bufx, bufy, acc):
            gbase = gi * 8
            rlo = jnp.maximum(lo, gbase)
            rhi = jnp.minimum(hi, gbase + 8)

            def row_body(r, carry):
                @pl.when((((r + 14) & 7) == 0) & ((r + 14) // 8 > pmax)
                         & (r + 14 <= H - 8) & (r + 4 < hi))
                def _():
                    for d in ring_dma(r + 14):
                        d.start()

                @pl.when((((r + 10) & 7) == 0) & ((r + 10) // 8 > pmax)
                         & (r + 10 <= H - 8))
                def _():
                    for d in ring_dma(r + 10):
                        d.wait()

                rloc = r - gbase
                slot_r = r & (RING - 1)

                def grp(j, a):
                    a0, a1, a2 = a
                    jcol = j * L
                    sg = gtb[slot_r, pl.ds(sloc + jcol, L)]
                    sp = prb[slot_r, pl.ds(sloc + jcol, L)]
                    sgx = (1.0 + TOL) * sg
                    for c in range(C):
                        gxv = bufx[c, rloc, pl.ds(jcol, L)]
                        gyv = bufy[c, rloc, pl.ds(jcol, L)]
                        lslot = gyv & (RING - 1)
                        lcol = gxv - cs
                        tg = plsc.load_gather(gtb, [lslot, lcol])
                        tp = plsc.load_gather(prb, [lslot, lcol])
                        m, sf, q = _softplus_terms(tg, tp, sg, sgx, sp)
                        a0 = a0 + m
                        a1 = a1 + sf
                        a2 = a2 + q
                    return a0, a1, a2

                return plsc.parallel_loop(0, JG, unroll=5, carry=carry)(grp)

            return lax.fori_loop(rlo, rhi, row_body, acc)

        idx_start(glo, gxa, gya, sema)
        for gp in range(NGRP // 2):
            ga = glo + 2 * gp
            gb = ga + 1
            idx_start(gb, gxb, gyb, semb)
            idx_wait(ga, gxa, gya, sema)
            acc = process_group(ga, gxa, gya, acc)
            if gp < NGRP // 2 - 1:
                idx_start(ga + 2, gxa, gya, sema)
            idx_wait(gb, gxb, gyb, semb)
            acc = process_group(gb, gxb, gyb, acc)
        return acc

    acc = lax.fori_loop(0, 3, strip_pass, (zero, zero, zero))

    accb[pl.ds(0, L)] = acc[0]
    accb[pl.ds(L, L)] = acc[1]
    accb[pl.ds(2 * L, L)] = acc[2]
    pltpu.sync_copy(accb, out_hbm.at[pl.ds(wid * 3 * L, 3 * L)])


@functools.partial(
    pl.kernel,
    out_type=jax.ShapeDtypeStruct((NW * 3 * L,), jnp.float32),
    mesh=plsc.VectorSubcoreMesh(core_axis_name="c", subcore_axis_name="s"),
    compiler_params=pltpu.CompilerParams(needs_layout_passes=False),
    scratch_types=[
        pltpu.VMEM((RING, CW), jnp.float32),
        pltpu.VMEM((RING, CW), jnp.float32),
        pltpu.VMEM((C, 8, SW), jnp.int32),
        pltpu.VMEM((C, 8, SW), jnp.int32),
        pltpu.VMEM((C, 8, SW), jnp.int32),
        pltpu.VMEM((C, 8, SW), jnp.int32),
        pltpu.VMEM((3 * L,), jnp.float32),
        pltpu.SemaphoreType.DMA,
        pltpu.SemaphoreType.DMA,
        pltpu.SemaphoreType.DMA,
    ],
)
def _depth_loss_partials(gs_hbm, gt_hbm, pr_hbm, out_hbm, gtb, prb,
                         gxa, gya, gxb, gyb, accb, sema, semb, semg):
    _sc_body(gs_hbm, gt_hbm, pr_hbm, out_hbm, gtb, prb, gxa, gya,
             gxb, gyb, accb, sema, semb, semg)


def kernel(pred_depth, gt_depth, grid, grid_shift):
    gs4 = jnp.transpose(grid_shift, (0, 3, 1, 2))
    parts = _depth_loss_partials(gs4, gt_depth, pred_depth)
    parts = parts.reshape(NW, 3, L)
    n_nz = jnp.sum(parts[:, 0])
    s_soft = jnp.sum(parts[:, 1])
    s_sq = jnp.sum(parts[:, 2])
    total = jnp.float32(H * W * C)
    depth_loss = s_soft / jnp.maximum(n_nz, 1.0)
    depth_loss_sim = s_sq / jnp.maximum(total - n_nz, 1.0)
    return depth_loss + depth_loss_sim
